# baseline (speedup 1.0000x reference)
import functools

import jax
import jax.numpy as jnp
from jax import lax
from jax.experimental import pallas as pl
from jax.experimental.pallas import tpu as pltpu
from jax.experimental.pallas import tpu_sc as plsc

VOCAB = 100000
VOCAB_P = 102400
D = 64
N_TOK = 204800

NC, NS = 2, 16
CHUNKS, CW = 50, 128
VPAD = 100096
SLICE = VPAD // NS

PBV = 4000
PBP = 4096
NPB = VOCAB // PBV

RB = 10240
NB = VOCAB_P // RB


def _hist_body(story_hbm, out_hbm, idx_v, ones_v, zer_v, counts_sp, sem):
    c = lax.axis_index("c")
    s = lax.axis_index("s")

    def fill_ones(k, _):
        ones_v[k // 8, pl.ds((k % 8) * 16, 16)] = jnp.full((16,), 1.0, jnp.float32)
        return _

    lax.fori_loop(0, CHUNKS * CW // 16, fill_ones, None)

    def fill_zeros(k, _):
        zer_v[pl.ds(k * 16, 16)] = jnp.zeros((16,), jnp.float32)
        return _

    lax.fori_loop(0, SLICE // 16, fill_zeros, None)

    pltpu.sync_copy(zer_v, counts_sp.at[pl.ds(s * SLICE, SLICE)])
    pltpu.sync_copy(story_hbm.at[c * NS + s], idx_v)
    plsc.subcore_barrier()

    def scatter_start(j, _):
        pltpu.async_copy(ones_v.at[j], counts_sp.at[idx_v.at[j]], sem, add=True)
        return _

    lax.fori_loop(0, CHUNKS, scatter_start, None)

    def scatter_wait(j, _):
        pltpu.make_async_copy(ones_v.at[j], counts_sp.at[idx_v.at[j]], sem).wait()
        return _

    lax.fori_loop(0, CHUNKS, scatter_wait, None)
    plsc.subcore_barrier()

    pltpu.sync_copy(counts_sp.at[pl.ds(s * SLICE, SLICE)], zer_v)
    pltpu.sync_copy(zer_v, out_hbm.at[pl.ds(c * VPAD + s * SLICE, SLICE)])


@functools.cache
def _histogram():
    return pl.kernel(
        _hist_body,
        out_type=jax.ShapeDtypeStruct((NC * VPAD,), jnp.float32),
        mesh=plsc.VectorSubcoreMesh(
            core_axis_name="c", subcore_axis_name="s",
            num_cores=NC, num_subcores=NS,
        ),
        scratch_types=[
            pltpu.VMEM((CHUNKS, CW), jnp.int32),
            pltpu.VMEM((CHUNKS, CW), jnp.float32),
            pltpu.VMEM((SLICE,), jnp.float32),
            pltpu.VMEM_SHARED((VPAD,), jnp.float32),
            pltpu.SemaphoreType.DMA,
        ],
    )


def _prep(C1, C2, C3):
    call = jnp.concatenate([C1, C2, C3], axis=0).reshape(3 * NPB, PBV, D)

    def body(src_ref, tall):
        blk = src_ref[0]
        tall[0, :, pl.ds(0, PBV)] = jnp.transpose(blk.astype(jnp.bfloat16), (1, 0))
        tall[0, :, pl.ds(PBV, PBP - PBV)] = jnp.zeros((D, PBP - PBV), jnp.bfloat16)

    return pl.pallas_call(
        body,
        grid=(3, NPB),
        in_specs=[pl.BlockSpec((1, PBV, D), lambda k, i: (k * NPB + i, 0, 0))],
        out_specs=pl.BlockSpec((1, D, PBP), lambda k, i: (k, 0, i)),
        out_shape=jax.ShapeDtypeStruct((3, D, VOCAB_P), jnp.bfloat16),
        compiler_params=pltpu.CompilerParams(
            dimension_semantics=("arbitrary", "arbitrary"),
        ),
    )(call)


def _hops_body(cnt0, cnt1, t1, t2, t3, out, acc, urow, m, z, o, c1v, c2v):
    p = pl.program_id(0)
    i = pl.program_id(1)
    cnt = cnt0[0] + cnt1[0]

    @pl.when((p == 0) & (i == 0))
    def _():
        acc[...] = jnp.zeros_like(acc)

    @pl.when(p == 0)
    def _():
        blk = t1[0]
        c1v[:, pl.ds(i * RB, RB)] = blk
        acc[...] += lax.dot_general(
            cnt.astype(jnp.bfloat16), blk, (((1,), (1,)), ((), ())),
            preferred_element_type=jnp.float32,
        )

    @pl.when((p == 1) & (i == 0))
    def _():
        urow[...] = acc[...] * (1.0 / N_TOK)
        m[...] = jnp.full_like(m, -1e30)
        z[...] = jnp.zeros_like(z)
        o[...] = jnp.zeros_like(o)

    @pl.when((p == 2) & (i == 0))
    def _():
        urow[...] = urow[...] + o[...] / z[...]
        m[...] = jnp.full_like(m, -1e30)
        z[...] = jnp.zeros_like(z)
        o[...] = jnp.zeros_like(o)

    @pl.when((p == 1))
    def _():
        c2v[:, pl.ds(i * RB, RB)] = t2[0]

    @pl.when(p >= 1)
    def _():
        a_blk = jnp.where(
            p == 1, c1v[:, pl.ds(i * RB, RB)], c2v[:, pl.ds(i * RB, RB)]
        )
        v_blk = jnp.where(p == 1, t2[0], t3[0])
        t = lax.dot_general(
            urow[...].astype(jnp.bfloat16), a_blk, (((1,), (0,)), ((), ())),
            preferred_element_type=jnp.float32,
        )
        m_old = m[0, 0]
        m_new = jnp.maximum(m_old, jnp.max(t))
        alpha = jnp.exp(m_old - m_new)
        w = cnt * jnp.exp(t - m_new)
        z[...] = z[...] * alpha + jnp.sum(w)
        o[...] = o[...] * alpha + lax.dot_general(
            w.astype(jnp.bfloat16), v_blk, (((1,), (1,)), ((), ())),
            preferred_element_type=jnp.float32,
        )
        m[...] = jnp.full_like(m, m_new)

    @pl.when((p == 2) & (i == NB - 1))
    def _():
        out[...] = urow[...] + o[...] / z[...]


def _hops(cnt0, cnt1, T1, T2, T3):
    return pl.pallas_call(
        _hops_body,
        grid=(3, NB),
        in_specs=[
            pl.BlockSpec((1, 1, RB), lambda p, i: (i, 0, 0)),
            pl.BlockSpec((1, 1, RB), lambda p, i: (i, 0, 0)),
            pl.BlockSpec(
                (1, D, RB),
                lambda p, i: (0, 0, jnp.where(p == 0, i, NB - 1)),
            ),
            pl.BlockSpec(
                (1, D, RB),
                lambda p, i: (1, 0, jnp.where(p == 1, i, jnp.where(p < 1, 0, NB - 1))),
            ),
            pl.BlockSpec(
                (1, D, RB),
                lambda p, i: (2, 0, jnp.where(p == 2, i, 0)),
            ),
        ],
        out_specs=pl.BlockSpec((1, D), lambda p, i: (0, 0)),
        out_shape=jax.ShapeDtypeStruct((1, D), jnp.float32),
        scratch_shapes=[
            pltpu.VMEM((1, D), jnp.float32),
            pltpu.VMEM((1, D), jnp.float32),
            pltpu.VMEM((1, 1), jnp.float32),
            pltpu.VMEM((1, 1), jnp.float32),
            pltpu.VMEM((1, D), jnp.float32),
            pltpu.VMEM((D, VOCAB_P), jnp.bfloat16),
            pltpu.VMEM((D, VOCAB_P), jnp.bfloat16),
        ],
        compiler_params=pltpu.CompilerParams(
            dimension_semantics=("arbitrary", "arbitrary"),
        ),
    )(cnt0, cnt1, T1, T2, T3)


@jax.jit
def kernel(story, C0, C1, C2, C3):
    del C0
    story_r = story.reshape(NC * NS, CHUNKS, CW)
    counts2 = _histogram()(story_r)
    Tall = _prep(C1, C2, C3)
    cnt = counts2.reshape(NC, VPAD)[:, :VOCAB].reshape(NC, NPB, PBV)
    cnt = jnp.pad(cnt, ((0, 0), (0, 0), (0, PBP - PBV)))
    cnt = cnt.reshape(NC, NB, 1, RB)
    return _hops(cnt[0], cnt[1], Tall, Tall, Tall)


# speedup vs baseline: 2.2602x; 2.2602x over previous
import functools

import jax
import jax.numpy as jnp
from jax import lax
from jax.experimental import pallas as pl
from jax.experimental.pallas import tpu as pltpu
from jax.experimental.pallas import tpu_sc as plsc

VOCAB = 100000
VOCAB_P = 102400
D = 64
N_TOK = 204800

NC, NS = 2, 16
CHUNKS, CW = 50, 128
VPAD = 100096
SLICE = VPAD // NS

PBV = 4000
PBP = 4096
NPB = VOCAB // PBV

RB = 10240
NB = VOCAB_P // RB


def _hist_body(story_hbm, out_hbm, idx_v, ones_v, zer_v, counts_sp, sem):
    c = lax.axis_index("c")
    s = lax.axis_index("s")

    def fill_ones(k, _):
        ones_v[k // 8, pl.ds((k % 8) * 16, 16)] = jnp.full((16,), 1.0, jnp.float32)
        return _

    lax.fori_loop(0, CHUNKS * CW // 16, fill_ones, None)

    def fill_zeros(k, _):
        zer_v[pl.ds(k * 16, 16)] = jnp.zeros((16,), jnp.float32)
        return _

    lax.fori_loop(0, SLICE // 16, fill_zeros, None)

    pltpu.sync_copy(zer_v, counts_sp.at[pl.ds(s * SLICE, SLICE)])
    pltpu.sync_copy(story_hbm.at[c * NS + s], idx_v)
    plsc.subcore_barrier()

    def scatter_start(j, _):
        pltpu.async_copy(ones_v.at[j], counts_sp.at[idx_v.at[j]], sem, add=True)
        return _

    lax.fori_loop(0, CHUNKS, scatter_start, None)

    def scatter_wait(j, _):
        pltpu.make_async_copy(ones_v.at[j], counts_sp.at[idx_v.at[j]], sem).wait()
        return _

    lax.fori_loop(0, CHUNKS, scatter_wait, None)
    plsc.subcore_barrier()

    pltpu.sync_copy(counts_sp.at[pl.ds(s * SLICE, SLICE)], zer_v)
    pltpu.sync_copy(zer_v, out_hbm.at[pl.ds(c * VPAD + s * SLICE, SLICE)])


@functools.cache
def _histogram():
    return pl.kernel(
        _hist_body,
        out_type=jax.ShapeDtypeStruct((NC * VPAD,), jnp.float32),
        mesh=plsc.VectorSubcoreMesh(
            core_axis_name="c", subcore_axis_name="s",
            num_cores=NC, num_subcores=NS,
        ),
        scratch_types=[
            pltpu.VMEM((CHUNKS, CW), jnp.int32),
            pltpu.VMEM((CHUNKS, CW), jnp.float32),
            pltpu.VMEM((SLICE,), jnp.float32),
            pltpu.VMEM_SHARED((VPAD,), jnp.float32),
            pltpu.SemaphoreType.DMA,
        ],
    )


def _prep(C1, C2, C3):
    def body(c1, c2, c3, t1, t2, t3):
        for src, dst in ((c1, t1), (c2, t2), (c3, t3)):
            blk = src[0]
            dst[:, pl.ds(0, PBV)] = jnp.transpose(blk.astype(jnp.bfloat16), (1, 0))
            dst[:, pl.ds(PBV, PBP - PBV)] = jnp.zeros((D, PBP - PBV), jnp.bfloat16)

    return pl.pallas_call(
        body,
        grid=(NPB,),
        in_specs=[pl.BlockSpec((1, PBV, D), lambda i: (i, 0, 0))] * 3,
        out_specs=[pl.BlockSpec((D, PBP), lambda i: (0, i))] * 3,
        out_shape=[jax.ShapeDtypeStruct((D, VOCAB_P), jnp.bfloat16)] * 3,
        compiler_params=pltpu.CompilerParams(
            dimension_semantics=("arbitrary",),
        ),
    )(
        C1.reshape(NPB, PBV, D),
        C2.reshape(NPB, PBV, D),
        C3.reshape(NPB, PBV, D),
    )


def _hops_body(cnt0, cnt1, t1, t2, t3, out, acc, urow, m, z, o, c1v, c2v):
    p = pl.program_id(0)
    i = pl.program_id(1)
    cnt = cnt0[0] + cnt1[0]

    @pl.when((p == 0) & (i == 0))
    def _():
        acc[...] = jnp.zeros_like(acc)

    @pl.when(p == 0)
    def _():
        blk = t1[...]
        c1v[:, pl.ds(i * RB, RB)] = blk
        acc[...] += lax.dot_general(
            cnt.astype(jnp.bfloat16), blk, (((1,), (1,)), ((), ())),
            preferred_element_type=jnp.float32,
        )

    @pl.when((p == 1) & (i == 0))
    def _():
        urow[...] = acc[...] * (1.0 / N_TOK)
        m[...] = jnp.full_like(m, -1e30)
        z[...] = jnp.zeros_like(z)
        o[...] = jnp.zeros_like(o)

    @pl.when((p == 2) & (i == 0))
    def _():
        urow[...] = urow[...] + o[...] / z[...]
        m[...] = jnp.full_like(m, -1e30)
        z[...] = jnp.zeros_like(z)
        o[...] = jnp.zeros_like(o)

    @pl.when((p == 1))
    def _():
        c2v[:, pl.ds(i * RB, RB)] = t2[...]

    @pl.when(p >= 1)
    def _():
        a_blk = jnp.where(
            p == 1, c1v[:, pl.ds(i * RB, RB)], c2v[:, pl.ds(i * RB, RB)]
        )
        v_blk = jnp.where(p == 1, t2[...], t3[...])
        t = lax.dot_general(
            urow[...].astype(jnp.bfloat16), a_blk, (((1,), (0,)), ((), ())),
            preferred_element_type=jnp.float32,
        )
        m_old = m[0, 0]
        m_new = jnp.maximum(m_old, jnp.max(t))
        alpha = jnp.exp(m_old - m_new)
        w = cnt * jnp.exp(t - m_new)
        z[...] = z[...] * alpha + jnp.sum(w)
        o[...] = o[...] * alpha + lax.dot_general(
            w.astype(jnp.bfloat16), v_blk, (((1,), (1,)), ((), ())),
            preferred_element_type=jnp.float32,
        )
        m[...] = jnp.full_like(m, m_new)

    @pl.when((p == 2) & (i == NB - 1))
    def _():
        out[...] = urow[...] + o[...] / z[...]


def _hops(cnt0, cnt1, T1, T2, T3):
    return pl.pallas_call(
        _hops_body,
        grid=(3, NB),
        in_specs=[
            pl.BlockSpec((1, 1, RB), lambda p, i: (i, 0, 0)),
            pl.BlockSpec((1, 1, RB), lambda p, i: (i, 0, 0)),
            pl.BlockSpec((D, RB), lambda p, i: (0, jnp.where(p == 0, i, NB - 1))),
            pl.BlockSpec(
                (D, RB),
                lambda p, i: (0, jnp.where(p == 1, i, jnp.where(p < 1, 0, NB - 1))),
            ),
            pl.BlockSpec((D, RB), lambda p, i: (0, jnp.where(p == 2, i, 0))),
        ],
        out_specs=pl.BlockSpec((1, D), lambda p, i: (0, 0)),
        out_shape=jax.ShapeDtypeStruct((1, D), jnp.float32),
        scratch_shapes=[
            pltpu.VMEM((1, D), jnp.float32),
            pltpu.VMEM((1, D), jnp.float32),
            pltpu.VMEM((1, 1), jnp.float32),
            pltpu.VMEM((1, 1), jnp.float32),
            pltpu.VMEM((1, D), jnp.float32),
            pltpu.VMEM((D, VOCAB_P), jnp.bfloat16),
            pltpu.VMEM((D, VOCAB_P), jnp.bfloat16),
        ],
        compiler_params=pltpu.CompilerParams(
            dimension_semantics=("arbitrary", "arbitrary"),
        ),
    )(cnt0, cnt1, T1, T2, T3)


@jax.jit
def kernel(story, C0, C1, C2, C3):
    del C0
    story_r = story.reshape(NC * NS, CHUNKS, CW)
    counts2 = _histogram()(story_r)
    T1, T2, T3 = _prep(C1, C2, C3)
    cnt = counts2.reshape(NC, VPAD)[:, :VOCAB].reshape(NC, NPB, PBV)
    cnt = jnp.pad(cnt, ((0, 0), (0, 0), (0, PBP - PBV)))
    cnt = cnt.reshape(NC, NB, 1, RB)
    return _hops(cnt[0], cnt[1], T1, T2, T3)
